# baseline (speedup 1.0000x reference)
import jax
import jax.numpy as jnp
import numpy as np
from jax.experimental import pallas as pl
from jax.experimental.pallas import tpu as pltpu

B = 32
H = 16
KV_LEN = 4096
KV_LORA_RANK = 512
QK_ROPE_HEAD_DIM = 64
V_HEAD_DIM = 128
D_MODEL = 4096
D_LAT = KV_LORA_RANK + QK_ROPE_HEAD_DIM
SCALE = 1.0 / np.sqrt(128.0 + 64.0)

N_COL_CHUNKS = 4
COL_CHUNK = D_MODEL // N_COL_CHUNKS
N_STEPS = B + N_COL_CHUNKS


def _mla_kernel(qn_ref, qp_ref, lat_ref, rope_ref, w_uv_ref, w_o_ref,
                out_ref, olat_scr):
    i = pl.program_id(0)

    @pl.when(i < B)
    def _attention():
        qn = qn_ref[0].astype(jnp.bfloat16)
        qp = qp_ref[0].astype(jnp.bfloat16)
        lat = lat_ref[0].astype(jnp.bfloat16)
        rope = rope_ref[0].astype(jnp.bfloat16)

        s = jax.lax.dot_general(
            qn, lat, (((1,), (0,)), ((), ())),
            preferred_element_type=jnp.float32,
        ) + jax.lax.dot_general(
            qp, rope, (((1,), (0,)), ((), ())),
            preferred_element_type=jnp.float32,
        )
        s = s * SCALE
        m = jnp.max(s, axis=-1, keepdims=True)
        p_f32 = jnp.exp(s - m)
        p = p_f32.astype(jnp.bfloat16)
        denom = jnp.sum(p_f32, axis=-1, keepdims=True)

        o_lat = jax.lax.dot_general(
            p, lat, (((1,), (1,)), ((), ())),
            preferred_element_type=jnp.float32,
        ) / denom
        olat_scr[pl.ds(i, 1)] = o_lat[None]

    @pl.when(i >= B)
    def _projection():
        o = jax.lax.dot_general(
            olat_scr[...].astype(jnp.bfloat16),
            w_uv_ref[...].astype(jnp.bfloat16),
            (((2,), (1,)), ((1,), (0,))),
            preferred_element_type=jnp.float32,
        )
        o = o.transpose(1, 0, 2).reshape(B, H * V_HEAD_DIM)
        out_ref[...] = jax.lax.dot_general(
            o.astype(jnp.bfloat16), w_o_ref[...].astype(jnp.bfloat16),
            (((1,), (0,)), ((), ())),
            preferred_element_type=jnp.float32,
        )


@jax.jit
def kernel(q_nope, q_pe, kv_cache, w_uv, w_o):
    kv_t = jnp.transpose(kv_cache, (0, 2, 1))

    last_b = B - 1
    return pl.pallas_call(
        _mla_kernel,
        grid=(N_STEPS,),
        in_specs=[
            pl.BlockSpec((1, H, KV_LORA_RANK),
                         lambda i: (jnp.minimum(i, last_b), 0, 0)),
            pl.BlockSpec((1, H, QK_ROPE_HEAD_DIM),
                         lambda i: (jnp.minimum(i, last_b), 0, 0)),
            pl.BlockSpec((1, KV_LORA_RANK, KV_LEN),
                         lambda i: (jnp.minimum(i, last_b), 0, 0)),
            pl.BlockSpec((1, QK_ROPE_HEAD_DIM, KV_LEN),
                         lambda i: (jnp.minimum(i, last_b),
                                    KV_LORA_RANK // QK_ROPE_HEAD_DIM, 0)),
            pl.BlockSpec((H, KV_LORA_RANK, V_HEAD_DIM),
                         lambda i: (0, 0, 0)),
            pl.BlockSpec((H * V_HEAD_DIM, COL_CHUNK),
                         lambda i: (0, jnp.clip(i - B, 0, N_COL_CHUNKS - 1))),
        ],
        out_specs=pl.BlockSpec(
            (B, COL_CHUNK),
            lambda i: (0, jnp.clip(i - B, 0, N_COL_CHUNKS - 1))),
        out_shape=jax.ShapeDtypeStruct((B, D_MODEL), jnp.float32),
        scratch_shapes=[pltpu.VMEM((B, H, KV_LORA_RANK), jnp.float32)],
        compiler_params=pltpu.CompilerParams(
            dimension_semantics=("arbitrary",),
        ),
    )(q_nope, q_pe, kv_t, kv_t, w_uv, w_o)


# speedup vs baseline: 1.0173x; 1.0173x over previous
import jax
import jax.numpy as jnp
import numpy as np
from jax.experimental import pallas as pl
from jax.experimental.pallas import tpu as pltpu

B = 32
H = 16
KV_LEN = 4096
KV_LORA_RANK = 512
QK_ROPE_HEAD_DIM = 64
V_HEAD_DIM = 128
D_MODEL = 4096
D_LAT = KV_LORA_RANK + QK_ROPE_HEAD_DIM
SCALE = 1.0 / np.sqrt(128.0 + 64.0)

B_BLK = 2
N_ATT = B // B_BLK
N_COL_CHUNKS = 8
COL_CHUNK = D_MODEL // N_COL_CHUNKS
N_STEPS = N_ATT + N_COL_CHUNKS


def _mla_kernel(qn_ref, qp_ref, lat_ref, rope_ref, w_uv_ref, w_o_ref,
                out_ref, olat_scr, o_scr):
    i = pl.program_id(0)

    @pl.when(i < N_ATT)
    def _attention():
        qn = qn_ref[...].astype(jnp.bfloat16)
        qp = qp_ref[...].astype(jnp.bfloat16)
        lat = lat_ref[...].astype(jnp.bfloat16)
        rope = rope_ref[...].astype(jnp.bfloat16)

        s = jax.lax.dot_general(
            qn, lat, (((2,), (1,)), ((0,), (0,))),
            preferred_element_type=jnp.float32,
        ) + jax.lax.dot_general(
            qp, rope, (((2,), (1,)), ((0,), (0,))),
            preferred_element_type=jnp.float32,
        )
        s = s * SCALE
        m = jnp.max(s, axis=-1, keepdims=True)
        p_f32 = jnp.exp(s - m)
        p = p_f32.astype(jnp.bfloat16)
        denom = jnp.sum(p_f32, axis=-1, keepdims=True)

        o_lat = jax.lax.dot_general(
            p, lat, (((2,), (2,)), ((0,), (0,))),
            preferred_element_type=jnp.float32,
        ) / denom
        olat_scr[pl.ds(i * B_BLK, B_BLK)] = o_lat

    @pl.when(i == N_ATT)
    def _up_project():
        o = jax.lax.dot_general(
            olat_scr[...].astype(jnp.bfloat16),
            w_uv_ref[...].astype(jnp.bfloat16),
            (((2,), (1,)), ((1,), (0,))),
            preferred_element_type=jnp.float32,
        )
        o_scr[...] = o.transpose(1, 0, 2).reshape(
            B, H * V_HEAD_DIM).astype(jnp.bfloat16)

    @pl.when(i >= N_ATT)
    def _projection():
        out_ref[...] = jax.lax.dot_general(
            o_scr[...], w_o_ref[...].astype(jnp.bfloat16),
            (((1,), (0,)), ((), ())),
            preferred_element_type=jnp.float32,
        )


@jax.jit
def kernel(q_nope, q_pe, kv_cache, w_uv, w_o):
    kv_t = jnp.transpose(kv_cache, (0, 2, 1))

    last_b = N_ATT - 1
    return pl.pallas_call(
        _mla_kernel,
        grid=(N_STEPS,),
        in_specs=[
            pl.BlockSpec((B_BLK, H, KV_LORA_RANK),
                         lambda i: (jnp.minimum(i, last_b), 0, 0)),
            pl.BlockSpec((B_BLK, H, QK_ROPE_HEAD_DIM),
                         lambda i: (jnp.minimum(i, last_b), 0, 0)),
            pl.BlockSpec((B_BLK, KV_LORA_RANK, KV_LEN),
                         lambda i: (jnp.minimum(i, last_b), 0, 0)),
            pl.BlockSpec((B_BLK, QK_ROPE_HEAD_DIM, KV_LEN),
                         lambda i: (jnp.minimum(i, last_b),
                                    KV_LORA_RANK // QK_ROPE_HEAD_DIM, 0)),
            pl.BlockSpec((H, KV_LORA_RANK, V_HEAD_DIM),
                         lambda i: (0, 0, 0)),
            pl.BlockSpec((H * V_HEAD_DIM, COL_CHUNK),
                         lambda i: (0, jnp.clip(i - N_ATT, 0, N_COL_CHUNKS - 1))),
        ],
        out_specs=pl.BlockSpec(
            (B, COL_CHUNK),
            lambda i: (0, jnp.clip(i - N_ATT, 0, N_COL_CHUNKS - 1))),
        out_shape=jax.ShapeDtypeStruct((B, D_MODEL), jnp.float32),
        scratch_shapes=[
            pltpu.VMEM((B, H, KV_LORA_RANK), jnp.float32),
            pltpu.VMEM((B, H * V_HEAD_DIM), jnp.bfloat16),
        ],
        compiler_params=pltpu.CompilerParams(
            dimension_semantics=("arbitrary",),
        ),
    )(q_nope, q_pe, kv_t, kv_t, w_uv, w_o)
